# baseline (speedup 1.0000x reference)
import functools

import jax
import jax.numpy as jnp
from jax import lax
from jax.experimental import pallas as pl
from jax.experimental.pallas import tpu as pltpu
from jax.experimental.pallas import tpu_sc as plsc

N_NODES = 10000
D = 128
N_EDGES = 320000

NC = 2
NS = 16
NW = NC * NS
K = 128
NCHUNK = 80
EPW = NCHUNK * K
E_PAD = NW * EPW
ROWS_PER_TILE = 632
N_PAD = ROWS_PER_TILE * NS


def _mm_body(x_ref, w_ref, o_ref):
    o_ref[...] = jnp.dot(x_ref[...], w_ref[...],
                         preferred_element_type=jnp.float32)


def _matmul(x, w):
    bm = 1000
    return pl.pallas_call(
        _mm_body,
        grid=(N_NODES // bm,),
        in_specs=[pl.BlockSpec((bm, D), lambda i: (i, 0)),
                  pl.BlockSpec((D, D), lambda i: (0, 0))],
        out_specs=pl.BlockSpec((bm, D), lambda i: (i, 0)),
        out_shape=jax.ShapeDtypeStruct((N_NODES, D), jnp.float32),
    )(x, w)


@functools.partial(
    pl.kernel,
    out_type=jax.ShapeDtypeStruct((NC, N_PAD, D), jnp.float32),
    mesh=plsc.VectorSubcoreMesh(core_axis_name="c", subcore_axis_name="s",
                                num_cores=NC, num_subcores=NS),
    scratch_types=[
        pltpu.VMEM((NCHUNK, K), jnp.int32),
        pltpu.VMEM((NCHUNK, K), jnp.int32),
        pltpu.VMEM((K, D), jnp.float32),
        pltpu.VMEM_SHARED((N_PAD, D), jnp.float32),
        pltpu.SemaphoreType.DMA,
    ],
)
def _sc_scatter(mat_hbm, src_hbm, dst_hbm, zero_hbm, out_hbm,
                src_v, dst_v, rows_v, acc, sem):
    cid = lax.axis_index("c")
    sid = lax.axis_index("s")
    wid = sid * NC + cid

    pltpu.sync_copy(src_hbm.at[wid], src_v)
    pltpu.sync_copy(dst_hbm.at[wid], dst_v)
    r0 = sid * ROWS_PER_TILE
    pltpu.sync_copy(zero_hbm.at[pl.ds(r0, ROWS_PER_TILE)],
                    acc.at[pl.ds(r0, ROWS_PER_TILE)])
    plsc.subcore_barrier()

    def body(j, carry):
        pltpu.async_copy(mat_hbm.at[src_v.at[j]], rows_v, sem).wait()
        pltpu.sync_copy(rows_v, acc.at[dst_v.at[j]], add=True)
        return carry

    lax.fori_loop(0, NCHUNK, body, 0)

    plsc.subcore_barrier()
    pltpu.sync_copy(acc.at[pl.ds(r0, ROWS_PER_TILE)],
                    out_hbm.at[cid, pl.ds(r0, ROWS_PER_TILE)])


def _comb_body(p_ref, b_ref, o_ref):
    o_ref[...] = p_ref[0] + p_ref[1] + b_ref[...]


def _combine(p, b2):
    bm = 1000
    return pl.pallas_call(
        _comb_body,
        grid=(N_NODES // bm,),
        in_specs=[pl.BlockSpec((NC, bm, D), lambda i: (0, i, 0)),
                  pl.BlockSpec((1, D), lambda i: (0, 0))],
        out_specs=pl.BlockSpec((bm, D), lambda i: (i, 0)),
        out_shape=jax.ShapeDtypeStruct((N_NODES, D), jnp.float32),
    )(p, b2)


def kernel(input, edge_index, W, b):
    mat = _matmul(input, W)
    npad_e = E_PAD - N_EDGES
    src = jnp.concatenate(
        [edge_index[0], jnp.zeros((npad_e,), jnp.int32)]
    ).reshape(NW, NCHUNK, K)
    dst = jnp.concatenate(
        [edge_index[1], jnp.full((npad_e,), N_PAD - 1, jnp.int32)]
    ).reshape(NW, NCHUNK, K)
    zeros = jnp.zeros((N_PAD, D), jnp.float32)
    partials = _sc_scatter(mat, src, dst, zeros)
    return _combine(partials, b.reshape(1, D))


# speedup vs baseline: 2.7195x; 2.7195x over previous
import functools

import jax
import jax.numpy as jnp
from jax import lax
from jax.experimental import pallas as pl
from jax.experimental.pallas import tpu as pltpu
from jax.experimental.pallas import tpu_sc as plsc

N_NODES = 10000
D = 128
N_EDGES = 320000

NC = 2
NS = 16
NW = NC * NS
EPW = N_EDGES // NW
K = 80
NCHUNK = EPW // K
ROWS_PER_TILE = 632
N_PAD = ROWS_PER_TILE * NS


def _mm_body(x_ref, w_ref, o_ref):
    o_ref[...] = jnp.dot(x_ref[...], w_ref[...],
                         preferred_element_type=jnp.float32)


def _matmul(x, w):
    bm = 1000
    return pl.pallas_call(
        _mm_body,
        grid=(N_NODES // bm,),
        in_specs=[pl.BlockSpec((bm, D), lambda i: (i, 0)),
                  pl.BlockSpec((D, D), lambda i: (0, 0))],
        out_specs=pl.BlockSpec((bm, D), lambda i: (i, 0)),
        out_shape=jax.ShapeDtypeStruct((N_NODES, D), jnp.float32),
    )(x, w)


@functools.partial(
    pl.kernel,
    out_type=jax.ShapeDtypeStruct((NC, N_PAD, D), jnp.float32),
    mesh=plsc.VectorSubcoreMesh(core_axis_name="c", subcore_axis_name="s",
                                num_cores=NC, num_subcores=NS),
    scratch_types=[
        pltpu.VMEM((25, K), jnp.int32),
        pltpu.VMEM((NCHUNK, K), jnp.int32),
        pltpu.VMEM((K, D), jnp.float32),
        pltpu.VMEM((K, D), jnp.float32),
        pltpu.VMEM_SHARED((N_PAD, D), jnp.float32),
        pltpu.SemaphoreType.DMA,
        pltpu.SemaphoreType.DMA,
    ],
)
def _sc_scatter(mat_hbm, src_hbm, dst_hbm, zero_hbm, out_hbm,
                src_v, dst_v, rows_a, rows_b, acc, sem_a, sem_b):
    cid = lax.axis_index("c")
    sid = lax.axis_index("s")
    wid = sid * NC + cid

    pltpu.sync_copy(dst_hbm.at[wid], dst_v)
    r0 = sid * ROWS_PER_TILE
    pltpu.sync_copy(zero_hbm.at[pl.ds(r0, ROWS_PER_TILE)],
                    acc.at[pl.ds(r0, ROWS_PER_TILE)])
    plsc.subcore_barrier()

    def blk_body(blk, carry):
        pltpu.sync_copy(src_hbm.at[wid, blk], src_v)
        j0 = blk * 25

        def pair(t, c2):
            ja = 2 * t
            jb = ja + 1
            cpa = pltpu.async_copy(mat_hbm.at[src_v.at[ja]], rows_a, sem_a)
            cpb = pltpu.async_copy(mat_hbm.at[src_v.at[jb]], rows_b, sem_b)
            cpa.wait()
            pltpu.sync_copy(rows_a, acc.at[dst_v.at[j0 + ja]], add=True)
            cpb.wait()
            pltpu.sync_copy(rows_b, acc.at[dst_v.at[j0 + jb]], add=True)
            return c2

        lax.fori_loop(0, 12, pair, 0)
        pltpu.async_copy(mat_hbm.at[src_v.at[24]], rows_a, sem_a).wait()
        pltpu.sync_copy(rows_a, acc.at[dst_v.at[j0 + 24]], add=True)
        return carry

    lax.fori_loop(0, 5, blk_body, 0)

    plsc.subcore_barrier()
    pltpu.sync_copy(acc.at[pl.ds(r0, ROWS_PER_TILE)],
                    out_hbm.at[cid, pl.ds(r0, ROWS_PER_TILE)])


def _comb_body(p_ref, b_ref, o_ref):
    o_ref[...] = p_ref[0] + p_ref[1] + b_ref[...]


def _combine(p, b2):
    bm = 1000
    return pl.pallas_call(
        _comb_body,
        grid=(N_NODES // bm,),
        in_specs=[pl.BlockSpec((NC, bm, D), lambda i: (0, i, 0)),
                  pl.BlockSpec((1, D), lambda i: (0, 0))],
        out_specs=pl.BlockSpec((bm, D), lambda i: (i, 0)),
        out_shape=jax.ShapeDtypeStruct((N_NODES, D), jnp.float32),
    )(p, b2)


def kernel(input, edge_index, W, b):
    mat = _matmul(input, W)
    src = edge_index[0].reshape(NW, 5, 25, K)
    dst = edge_index[1].reshape(NW, NCHUNK, K)
    zeros = jnp.zeros((N_PAD, D), jnp.float32)
    partials = _sc_scatter(mat, src, dst, zeros)
    return _combine(partials, b.reshape(1, D))


# speedup vs baseline: 2.7728x; 1.0196x over previous
import functools

import jax
import jax.numpy as jnp
from jax import lax
from jax.experimental import pallas as pl
from jax.experimental.pallas import tpu as pltpu
from jax.experimental.pallas import tpu_sc as plsc

N_NODES = 10000
D = 128
N_EDGES = 320000

NC = 2
NS = 16
NW = NC * NS
EPW = N_EDGES // NW
K = 80
NCHUNK = EPW // K
ROWS_PER_TILE = 632
N_PAD = ROWS_PER_TILE * NS


def _mm_body(x_ref, w_ref, o_ref):
    o_ref[...] = jnp.dot(x_ref[...], w_ref[...],
                         preferred_element_type=jnp.float32)


def _matmul(x, w):
    bm = 1000
    return pl.pallas_call(
        _mm_body,
        grid=(N_NODES // bm,),
        in_specs=[pl.BlockSpec((bm, D), lambda i: (i, 0)),
                  pl.BlockSpec((D, D), lambda i: (0, 0))],
        out_specs=pl.BlockSpec((bm, D), lambda i: (i, 0)),
        out_shape=jax.ShapeDtypeStruct((N_NODES, D), jnp.float32),
    )(x, w)


@functools.partial(
    pl.kernel,
    out_type=jax.ShapeDtypeStruct((NC, N_PAD, D), jnp.float32),
    mesh=plsc.VectorSubcoreMesh(core_axis_name="c", subcore_axis_name="s",
                                num_cores=NC, num_subcores=NS),
    scratch_types=[
        pltpu.VMEM((25, K), jnp.int32),
        pltpu.VMEM((25, K), jnp.int32),
        pltpu.VMEM((K, D), jnp.float32),
        pltpu.VMEM((K, D), jnp.float32),
        pltpu.VMEM((K, D), jnp.float32),
        pltpu.VMEM_SHARED((N_PAD, D), jnp.float32),
        pltpu.SemaphoreType.DMA,
        pltpu.SemaphoreType.DMA,
        pltpu.SemaphoreType.DMA,
    ],
)
def _sc_scatter(mat_hbm, src_hbm, dst_hbm, zero_hbm, out_hbm,
                src_v, dst_v, rows_a, rows_b, rows_c, acc,
                sem_a, sem_b, sem_c):
    cid = lax.axis_index("c")
    sid = lax.axis_index("s")
    wid = sid * NC + cid

    r0 = sid * ROWS_PER_TILE
    pltpu.sync_copy(zero_hbm.at[pl.ds(r0, ROWS_PER_TILE)],
                    acc.at[pl.ds(r0, ROWS_PER_TILE)])
    plsc.subcore_barrier()

    def blk_body(blk, carry):
        pltpu.sync_copy(src_hbm.at[wid, blk], src_v)
        pltpu.sync_copy(dst_hbm.at[wid, blk], dst_v)

        def triple(t, c2):
            ja = 3 * t
            jb = ja + 1
            jc = ja + 2
            cpa = pltpu.async_copy(mat_hbm.at[src_v.at[ja]], rows_a, sem_a)
            cpb = pltpu.async_copy(mat_hbm.at[src_v.at[jb]], rows_b, sem_b)
            cpc = pltpu.async_copy(mat_hbm.at[src_v.at[jc]], rows_c, sem_c)
            cpa.wait()
            pltpu.sync_copy(rows_a, acc.at[dst_v.at[ja]], add=True)
            cpb.wait()
            pltpu.sync_copy(rows_b, acc.at[dst_v.at[jb]], add=True)
            cpc.wait()
            pltpu.sync_copy(rows_c, acc.at[dst_v.at[jc]], add=True)
            return c2

        lax.fori_loop(0, 8, triple, 0)
        pltpu.async_copy(mat_hbm.at[src_v.at[24]], rows_a, sem_a).wait()
        pltpu.sync_copy(rows_a, acc.at[dst_v.at[24]], add=True)
        return carry

    lax.fori_loop(0, 5, blk_body, 0)

    plsc.subcore_barrier()
    pltpu.sync_copy(acc.at[pl.ds(r0, ROWS_PER_TILE)],
                    out_hbm.at[cid, pl.ds(r0, ROWS_PER_TILE)])


def _comb_body(p_ref, b_ref, o_ref):
    o_ref[...] = p_ref[0] + p_ref[1] + b_ref[...]


def _combine(p, b2):
    bm = 1000
    return pl.pallas_call(
        _comb_body,
        grid=(N_NODES // bm,),
        in_specs=[pl.BlockSpec((NC, bm, D), lambda i: (0, i, 0)),
                  pl.BlockSpec((1, D), lambda i: (0, 0))],
        out_specs=pl.BlockSpec((bm, D), lambda i: (i, 0)),
        out_shape=jax.ShapeDtypeStruct((N_NODES, D), jnp.float32),
    )(p, b2)


def kernel(input, edge_index, W, b):
    mat = _matmul(input, W)
    src = edge_index[0].reshape(NW, 5, 25, K)
    dst = edge_index[1].reshape(NW, 5, 25, K)
    zeros = jnp.zeros((N_PAD, D), jnp.float32)
    partials = _sc_scatter(mat, src, dst, zeros)
    return _combine(partials, b.reshape(1, D))


# speedup vs baseline: 2.8304x; 1.0208x over previous
import functools

import jax
import jax.numpy as jnp
from jax import lax
from jax.experimental import pallas as pl
from jax.experimental.pallas import tpu as pltpu
from jax.experimental.pallas import tpu_sc as plsc

N_NODES = 10000
D = 128
N_EDGES = 320000

NC = 2
NS = 16
NW = NC * NS
EPW = N_EDGES // NW
K = 80
NCHUNK = EPW // K
ROWS_PER_TILE = 632
N_PAD = ROWS_PER_TILE * NS


def _mm_body(x_ref, w_ref, o_ref):
    o_ref[...] = jnp.dot(x_ref[...], w_ref[...],
                         preferred_element_type=jnp.float32)


def _matmul(x, w):
    bm = 2000
    return pl.pallas_call(
        _mm_body,
        grid=(N_NODES // bm,),
        in_specs=[pl.BlockSpec((bm, D), lambda i: (i, 0)),
                  pl.BlockSpec((D, D), lambda i: (0, 0))],
        out_specs=pl.BlockSpec((bm, D), lambda i: (i, 0)),
        out_shape=jax.ShapeDtypeStruct((N_NODES, D), jnp.float32),
    )(x, w)


@functools.partial(
    pl.kernel,
    out_type=jax.ShapeDtypeStruct((NC, N_PAD, D), jnp.float32),
    mesh=plsc.VectorSubcoreMesh(core_axis_name="c", subcore_axis_name="s",
                                num_cores=NC, num_subcores=NS),
    scratch_types=[
        pltpu.VMEM((25, K), jnp.int32),
        pltpu.VMEM((25, K), jnp.int32),
        pltpu.VMEM((K, D), jnp.float32),
        pltpu.VMEM((K, D), jnp.float32),
        pltpu.VMEM((K, D), jnp.float32),
        pltpu.VMEM_SHARED((N_PAD, D), jnp.float32),
        pltpu.SemaphoreType.DMA,
        pltpu.SemaphoreType.DMA,
        pltpu.SemaphoreType.DMA,
    ],
)
def _sc_scatter(mat_hbm, src_hbm, dst_hbm, zero_hbm, out_hbm,
                src_v, dst_v, rows_a, rows_b, rows_c, acc,
                sem_a, sem_b, sem_c):
    cid = lax.axis_index("c")
    sid = lax.axis_index("s")
    wid = sid * NC + cid

    r0 = sid * ROWS_PER_TILE
    pltpu.sync_copy(zero_hbm.at[pl.ds(r0, ROWS_PER_TILE)],
                    acc.at[pl.ds(r0, ROWS_PER_TILE)])
    plsc.subcore_barrier()

    def blk_body(blk, carry):
        pltpu.sync_copy(src_hbm.at[wid, blk], src_v)
        pltpu.sync_copy(dst_hbm.at[wid, blk], dst_v)

        def triple(t, c2):
            ja = 3 * t
            jb = ja + 1
            jc = ja + 2
            cpa = pltpu.async_copy(mat_hbm.at[src_v.at[ja]], rows_a, sem_a)
            cpb = pltpu.async_copy(mat_hbm.at[src_v.at[jb]], rows_b, sem_b)
            cpc = pltpu.async_copy(mat_hbm.at[src_v.at[jc]], rows_c, sem_c)
            cpa.wait()
            pltpu.sync_copy(rows_a, acc.at[dst_v.at[ja]], add=True)
            cpb.wait()
            pltpu.sync_copy(rows_b, acc.at[dst_v.at[jb]], add=True)
            cpc.wait()
            pltpu.sync_copy(rows_c, acc.at[dst_v.at[jc]], add=True)
            return c2

        lax.fori_loop(0, 8, triple, 0)
        pltpu.async_copy(mat_hbm.at[src_v.at[24]], rows_a, sem_a).wait()
        pltpu.sync_copy(rows_a, acc.at[dst_v.at[24]], add=True)
        return carry

    lax.fori_loop(0, 5, blk_body, 0)

    plsc.subcore_barrier()
    pltpu.sync_copy(acc.at[pl.ds(r0, ROWS_PER_TILE)],
                    out_hbm.at[cid, pl.ds(r0, ROWS_PER_TILE)])


def _comb_body(p_ref, b_ref, o_ref):
    o_ref[...] = p_ref[0] + p_ref[1] + b_ref[...]


def _combine(p, b2):
    bm = 2000
    return pl.pallas_call(
        _comb_body,
        grid=(N_NODES // bm,),
        in_specs=[pl.BlockSpec((NC, bm, D), lambda i: (0, i, 0)),
                  pl.BlockSpec((1, D), lambda i: (0, 0))],
        out_specs=pl.BlockSpec((bm, D), lambda i: (i, 0)),
        out_shape=jax.ShapeDtypeStruct((N_NODES, D), jnp.float32),
    )(p, b2)


def kernel(input, edge_index, W, b):
    mat = _matmul(input, W)
    src = edge_index[0].reshape(NW, 5, 25, K)
    dst = edge_index[1].reshape(NW, 5, 25, K)
    zeros = jnp.zeros((N_PAD, D), jnp.float32)
    partials = _sc_scatter(mat, src, dst, zeros)
    return _combine(partials, b.reshape(1, D))


# speedup vs baseline: 3.7758x; 1.3340x over previous
import functools

import jax
import jax.numpy as jnp
from jax import lax
from jax.experimental import pallas as pl
from jax.experimental.pallas import tpu as pltpu
from jax.experimental.pallas import tpu_sc as plsc

N_NODES = 10000
D = 128
N_EDGES = 320000

NC = 2
NS = 16
NW = NC * NS
EPW = N_EDGES // NW
K = 80
NCHUNK = EPW // K
ROWS_PER_TILE = 632
N_PAD = ROWS_PER_TILE * NS


def _mm_body(x_ref, w_ref, o_ref):
    o_ref[...] = jnp.dot(x_ref[...], w_ref[...],
                         preferred_element_type=jnp.float32)


def _matmul(x, w):
    bm = 2000
    return pl.pallas_call(
        _mm_body,
        grid=(N_NODES // bm,),
        in_specs=[pl.BlockSpec((bm, D), lambda i: (i, 0)),
                  pl.BlockSpec((D, D), lambda i: (0, 0))],
        out_specs=pl.BlockSpec((bm, D), lambda i: (i, 0)),
        out_shape=jax.ShapeDtypeStruct((N_NODES, D), jnp.float32),
    )(x, w)


@functools.partial(
    pl.kernel,
    out_type=jax.ShapeDtypeStruct((NC, N_PAD, D), jnp.float32),
    mesh=plsc.VectorSubcoreMesh(core_axis_name="c", subcore_axis_name="s",
                                num_cores=NC, num_subcores=NS),
    scratch_types=[
        pltpu.VMEM((25, K), jnp.int32),
        pltpu.VMEM((25, K), jnp.int32),
        pltpu.VMEM((K, D), jnp.float32),
        pltpu.VMEM((K, D), jnp.float32),
        pltpu.VMEM((K, D), jnp.float32),
        pltpu.VMEM_SHARED((N_PAD, D), jnp.float32),
        pltpu.SemaphoreType.DMA,
        pltpu.SemaphoreType.DMA,
        pltpu.SemaphoreType.DMA,
    ],
)
def _sc_scatter(mat_hbm, src_hbm, dst_hbm, zero_hbm, out_hbm,
                src_v, dst_v, rows_a, rows_b, rows_c, acc,
                sem_a, sem_b, sem_c):
    cid = lax.axis_index("c")
    sid = lax.axis_index("s")
    wid = sid * NC + cid

    r0 = sid * ROWS_PER_TILE
    pltpu.sync_copy(zero_hbm.at[pl.ds(r0, ROWS_PER_TILE)],
                    acc.at[pl.ds(r0, ROWS_PER_TILE)])
    plsc.subcore_barrier()

    def blk_body(blk, carry):
        pltpu.sync_copy(src_hbm.at[wid, blk], src_v)
        pltpu.sync_copy(dst_hbm.at[wid, blk], dst_v)

        pltpu.async_copy(mat_hbm.at[src_v.at[0]], rows_a, sem_a)
        pltpu.async_copy(mat_hbm.at[src_v.at[1]], rows_b, sem_b)
        pltpu.async_copy(mat_hbm.at[src_v.at[2]], rows_c, sem_c)

        def rot(t, c2):
            ja = 3 * t
            jb = ja + 1
            jc = ja + 2
            pltpu.make_async_copy(mat_hbm.at[src_v.at[ja]], rows_a,
                                  sem_a).wait()
            pltpu.sync_copy(rows_a, acc.at[dst_v.at[ja]], add=True)
            pltpu.async_copy(mat_hbm.at[src_v.at[ja + 3]], rows_a, sem_a)
            pltpu.make_async_copy(mat_hbm.at[src_v.at[jb]], rows_b,
                                  sem_b).wait()
            pltpu.sync_copy(rows_b, acc.at[dst_v.at[jb]], add=True)
            pltpu.async_copy(mat_hbm.at[src_v.at[jb + 3]], rows_b, sem_b)
            pltpu.make_async_copy(mat_hbm.at[src_v.at[jc]], rows_c,
                                  sem_c).wait()
            pltpu.sync_copy(rows_c, acc.at[dst_v.at[jc]], add=True)
            pltpu.async_copy(mat_hbm.at[src_v.at[jc + 3]], rows_c, sem_c)
            return c2

        lax.fori_loop(0, 7, rot, 0)

        pltpu.make_async_copy(mat_hbm.at[src_v.at[21]], rows_a, sem_a).wait()
        pltpu.sync_copy(rows_a, acc.at[dst_v.at[21]], add=True)
        pltpu.async_copy(mat_hbm.at[src_v.at[24]], rows_a, sem_a)
        pltpu.make_async_copy(mat_hbm.at[src_v.at[22]], rows_b, sem_b).wait()
        pltpu.sync_copy(rows_b, acc.at[dst_v.at[22]], add=True)
        pltpu.make_async_copy(mat_hbm.at[src_v.at[23]], rows_c, sem_c).wait()
        pltpu.sync_copy(rows_c, acc.at[dst_v.at[23]], add=True)
        pltpu.make_async_copy(mat_hbm.at[src_v.at[24]], rows_a, sem_a).wait()
        pltpu.sync_copy(rows_a, acc.at[dst_v.at[24]], add=True)
        return carry

    lax.fori_loop(0, 5, blk_body, 0)

    plsc.subcore_barrier()
    pltpu.sync_copy(acc.at[pl.ds(r0, ROWS_PER_TILE)],
                    out_hbm.at[cid, pl.ds(r0, ROWS_PER_TILE)])


def _comb_body(p_ref, b_ref, o_ref):
    o_ref[...] = p_ref[0] + p_ref[1] + b_ref[...]


def _combine(p, b2):
    bm = 2000
    return pl.pallas_call(
        _comb_body,
        grid=(N_NODES // bm,),
        in_specs=[pl.BlockSpec((NC, bm, D), lambda i: (0, i, 0)),
                  pl.BlockSpec((1, D), lambda i: (0, 0))],
        out_specs=pl.BlockSpec((bm, D), lambda i: (i, 0)),
        out_shape=jax.ShapeDtypeStruct((N_NODES, D), jnp.float32),
    )(p, b2)


def kernel(input, edge_index, W, b):
    mat = _matmul(input, W)
    src = edge_index[0].reshape(NW, 5, 25, K)
    dst = edge_index[1].reshape(NW, 5, 25, K)
    zeros = jnp.zeros((N_PAD, D), jnp.float32)
    partials = _sc_scatter(mat, src, dst, zeros)
    return _combine(partials, b.reshape(1, D))
